# baseline (speedup 1.0000x reference)
import functools

import jax
import jax.numpy as jnp
from jax import lax
from jax.experimental import pallas as pl
from jax.experimental.pallas import tpu as pltpu
from jax.experimental.pallas import tpu_sc as plsc

T = 2048
D = 1024
E = 8
F = 2048
EPAD = 128
BT = 256
JMAX = T // BT
NSLOT = 2 * T + E * BT
NW = 32
SPT = NSLOT // NW
GCH = 32
TPT = T // NW
CCH = 32
NEG = -1e30


def _router_body(x_ref, g_ref, logits_ref, i1_ref, i2_ref, wa_ref, wb_ref,
                 xc_ref):
    x = x_ref[...]
    xc_ref[...] = x
    g = g_ref[...]
    logits = lax.dot_general(x, g, (((1,), (1,)), ((), ())),
                             preferred_element_type=jnp.float32)
    col = lax.broadcasted_iota(jnp.int32, (T, EPAD), 1)
    lg = jnp.where(col < E, logits, NEG)
    m1 = jnp.max(lg, axis=1, keepdims=True)
    i1 = jnp.min(jnp.where(lg == m1, col, EPAD), axis=1, keepdims=True)
    lg2 = jnp.where(col == i1, NEG, lg)
    m2 = jnp.max(lg2, axis=1, keepdims=True)
    i2 = jnp.min(jnp.where(lg2 == m2, col, EPAD), axis=1, keepdims=True)
    logits_ref[...] = logits
    i1_ref[...] = i1
    i2_ref[...] = i2
    wa_ref[...] = jax.nn.sigmoid(m1 - m2)
    wb_ref[...] = jax.nn.sigmoid(m2 - m1)


def _router(x, gate_pad):
    return pl.pallas_call(
        _router_body,
        out_shape=[
            jax.ShapeDtypeStruct((T, EPAD), jnp.float32),
            jax.ShapeDtypeStruct((T, 1), jnp.int32),
            jax.ShapeDtypeStruct((T, 1), jnp.int32),
            jax.ShapeDtypeStruct((T, 1), jnp.float32),
            jax.ShapeDtypeStruct((T, 1), jnp.float32),
            jax.ShapeDtypeStruct((T, D), jnp.float32),
        ],
    )(x, gate_pad)


def _dispatch(x, src_of_slot):
    mesh = plsc.VectorSubcoreMesh(core_axis_name="c", subcore_axis_name="s")

    @functools.partial(
        pl.kernel,
        out_type=jax.ShapeDtypeStruct((NSLOT, D), jnp.float32),
        mesh=mesh,
        scratch_types=[
            pltpu.VMEM((SPT,), jnp.int32),
            pltpu.VMEM((GCH, D), jnp.float32),
            pltpu.VMEM((GCH, D), jnp.float32),
            pltpu.SemaphoreType.DMA,
            pltpu.SemaphoreType.DMA,
            pltpu.SemaphoreType.DMA,
        ],
    )
    def k(x_hbm, src_hbm, xs_hbm, idx_v, buf0, buf1, gsem, wsem0, wsem1):
        w = lax.axis_index("s") * 2 + lax.axis_index("c")
        base = w * SPT
        pltpu.sync_copy(src_hbm.at[pl.ds(base, SPT)], idx_v)
        nch = SPT // GCH
        bufs = (buf0, buf1)
        wsems = (wsem0, wsem1)

        def gather(kk, buf):
            return pltpu.async_copy(
                x_hbm.at[idx_v.at[pl.ds(kk * GCH, GCH)]], buf, gsem)

        g = gather(0, bufs[0])
        for kk in range(nch):
            g.wait()
            if kk + 1 < nch:
                if kk >= 1:
                    pltpu.make_async_copy(
                        bufs[(kk + 1) % 2],
                        xs_hbm.at[pl.ds(base + (kk - 1) * GCH, GCH)],
                        wsems[(kk - 1) % 2]).wait()
                g = gather(kk + 1, bufs[(kk + 1) % 2])
            pltpu.async_copy(
                bufs[kk % 2],
                xs_hbm.at[pl.ds(base + kk * GCH, GCH)],
                wsems[kk % 2])
        pltpu.make_async_copy(
            bufs[(nch - 1) % 2],
            xs_hbm.at[pl.ds(base + (nch - 1) * GCH, GCH)],
            wsems[(nch - 1) % 2]).wait()
        pltpu.make_async_copy(
            bufs[(nch - 2) % 2],
            xs_hbm.at[pl.ds(base + (nch - 2) * GCH, GCH)],
            wsems[(nch - 2) % 2]).wait()

    return k(x, src_of_slot)


def _mlp_body(sc_ref, xs_ref, w1_ref, w3_ref, w2_ref, ws_ref, out_ref):
    e = pl.program_id(0)
    j = pl.program_id(1)

    @pl.when(j < sc_ref[e])
    def _():
        xb = xs_ref[...]
        a = lax.dot_general(xb, w1_ref[0], (((1,), (1,)), ((), ())),
                            preferred_element_type=jnp.float32)
        b = lax.dot_general(xb, w3_ref[0], (((1,), (1,)), ((), ())),
                            preferred_element_type=jnp.float32)
        h = a * jax.nn.sigmoid(a) * b
        o = lax.dot_general(h, w2_ref[0], (((1,), (1,)), ((), ())),
                            preferred_element_type=jnp.float32)
        out_ref[...] = o * ws_ref[0, 0, :][:, None]


def _mlp(scalars, xs, w1, w3, w2, w_slot3):
    def _rb(e, j, s):
        return s[E + e] + jnp.minimum(j, jnp.maximum(s[e] - 1, 0))

    grid_spec = pltpu.PrefetchScalarGridSpec(
        num_scalar_prefetch=1,
        grid=(E, JMAX),
        in_specs=[
            pl.BlockSpec((BT, D), lambda e, j, s: (_rb(e, j, s), 0)),
            pl.BlockSpec((1, F, D), lambda e, j, s: (e, 0, 0)),
            pl.BlockSpec((1, F, D), lambda e, j, s: (e, 0, 0)),
            pl.BlockSpec((1, D, F), lambda e, j, s: (e, 0, 0)),
            pl.BlockSpec((1, 1, BT), lambda e, j, s: (_rb(e, j, s), 0, 0)),
        ],
        out_specs=pl.BlockSpec((BT, D), lambda e, j, s: (_rb(e, j, s), 0)),
    )
    return pl.pallas_call(
        _mlp_body,
        grid_spec=grid_spec,
        out_shape=jax.ShapeDtypeStruct((NSLOT, D), jnp.float32),
        compiler_params=pltpu.CompilerParams(
            dimension_semantics=("arbitrary", "arbitrary")),
    )(scalars, xs, w1, w3, w2, w_slot3)


def _combine(buf, pos_a, pos_b):
    mesh = plsc.VectorSubcoreMesh(core_axis_name="c", subcore_axis_name="s")

    @functools.partial(
        pl.kernel,
        out_type=jax.ShapeDtypeStruct((T, D), jnp.float32),
        mesh=mesh,
        scratch_types=[
            pltpu.VMEM((TPT,), jnp.int32),
            pltpu.VMEM((TPT,), jnp.int32),
            pltpu.VMEM((CCH, D), jnp.float32),
            pltpu.VMEM((CCH, D), jnp.float32),
            pltpu.VMEM((CCH, D), jnp.float32),
            pltpu.SemaphoreType.DMA,
            pltpu.SemaphoreType.DMA,
        ],
    )
    def k(buf_hbm, pa_hbm, pb_hbm, out_hbm, ia_v, ib_v, a_v, b_v, o_v,
          sem, wsem):
        w = lax.axis_index("s") * 2 + lax.axis_index("c")
        base = w * TPT
        pltpu.sync_copy(pa_hbm.at[pl.ds(base, TPT)], ia_v)
        pltpu.sync_copy(pb_hbm.at[pl.ds(base, TPT)], ib_v)
        for kk in range(TPT // CCH):
            ca = pltpu.async_copy(
                buf_hbm.at[ia_v.at[pl.ds(kk * CCH, CCH)]], a_v, sem)
            cb = pltpu.async_copy(
                buf_hbm.at[ib_v.at[pl.ds(kk * CCH, CCH)]], b_v, sem)
            ca.wait()
            cb.wait()
            if kk > 0:
                pltpu.make_async_copy(
                    o_v, out_hbm.at[pl.ds(base + (kk - 1) * CCH, CCH)],
                    wsem).wait()

            @pl.loop(0, CCH)
            def _(r):
                for sg in range(D // 16):
                    sl = pl.ds(sg * 16, 16)
                    o_v[r, sl] = a_v[r, sl] + b_v[r, sl]

            pltpu.async_copy(
                o_v, out_hbm.at[pl.ds(base + kk * CCH, CCH)], wsem)
        pltpu.make_async_copy(
            o_v, out_hbm.at[pl.ds(base + (TPT // CCH - 1) * CCH, CCH)],
            wsem).wait()

    return k(buf, pos_a, pos_b)


def kernel(hidden_states, gate_w, w1, w2, w3):
    b, s, d = hidden_states.shape
    x = hidden_states.reshape(T, D)
    gate_pad = jnp.zeros((EPAD, D), jnp.float32).at[:E].set(gate_w)

    logits_pad, i1, i2, wa, wb, x_copy = _router(x, gate_pad)
    router_logits = logits_pad[:, :E]

    flat_e = jnp.stack([i1[:, 0], i2[:, 0]], axis=1).reshape(-1)
    onehot = (flat_e[:, None] == jnp.arange(E)[None, :]).astype(jnp.int32)
    csum = jnp.cumsum(onehot, axis=0)
    rank = jnp.take_along_axis(csum, flat_e[:, None], axis=1)[:, 0] - 1
    counts = csum[-1]
    nb = ((counts + BT - 1) // BT).astype(jnp.int32)
    cnb = jnp.concatenate([jnp.zeros((1,), jnp.int32),
                           jnp.cumsum(nb)[:-1].astype(jnp.int32)])
    scalars = jnp.concatenate([nb, cnb])
    pstart = cnb * BT
    pslot = (pstart[flat_e] + rank).astype(jnp.int32)
    src_tok = (jnp.arange(2 * T, dtype=jnp.int32) // 2)
    wflat = jnp.stack([wa[:, 0], wb[:, 0]], axis=1).reshape(-1)
    src_of_slot = jnp.zeros((NSLOT,), jnp.int32).at[pslot].set(src_tok)
    w_slot = jnp.zeros((NSLOT,), jnp.float32).at[pslot].set(wflat)
    w_slot3 = w_slot.reshape(NSLOT // BT, 1, BT)
    pos_a = pslot[0::2]
    pos_b = pslot[1::2]

    xs = _dispatch(x_copy, src_of_slot)
    out_buf = _mlp(scalars, xs, w1, w3, w2, w_slot3)
    final = _combine(out_buf, pos_a, pos_b)
    return final.reshape(b, s, d), router_logits


# speedup vs baseline: 1.3880x; 1.3880x over previous
import functools

import jax
import jax.numpy as jnp
from jax import lax
from jax.experimental import pallas as pl
from jax.experimental.pallas import tpu as pltpu
from jax.experimental.pallas import tpu_sc as plsc

T = 2048
D = 1024
E = 8
F = 2048
EPAD = 128
BT = 256
JMAX = T // BT
NSLOT = 2 * T + E * BT
NW = 32
SPT = NSLOT // NW
GCH = 32
TPT = T // NW
CCH = 32
NEG = -1e30


def _router_body(x_ref, g_ref, logits_ref, i1_ref, i2_ref, wa_ref, wb_ref,
                 xc_ref):
    x = x_ref[...]
    xc_ref[...] = x
    g = g_ref[...]
    logits = lax.dot_general(x, g, (((1,), (1,)), ((), ())),
                             preferred_element_type=jnp.float32)
    col = lax.broadcasted_iota(jnp.int32, (T, EPAD), 1)
    lg = jnp.where(col < E, logits, NEG)
    m1 = jnp.max(lg, axis=1, keepdims=True)
    i1 = jnp.min(jnp.where(lg == m1, col, EPAD), axis=1, keepdims=True)
    lg2 = jnp.where(col == i1, NEG, lg)
    m2 = jnp.max(lg2, axis=1, keepdims=True)
    i2 = jnp.min(jnp.where(lg2 == m2, col, EPAD), axis=1, keepdims=True)
    logits_ref[...] = logits
    i1_ref[...] = i1
    i2_ref[...] = i2
    wa_ref[...] = jax.nn.sigmoid(m1 - m2)
    wb_ref[...] = jax.nn.sigmoid(m2 - m1)


def _router(x, gate_pad):
    return pl.pallas_call(
        _router_body,
        out_shape=[
            jax.ShapeDtypeStruct((T, EPAD), jnp.float32),
            jax.ShapeDtypeStruct((T, 1), jnp.int32),
            jax.ShapeDtypeStruct((T, 1), jnp.int32),
            jax.ShapeDtypeStruct((T, 1), jnp.float32),
            jax.ShapeDtypeStruct((T, 1), jnp.float32),
            jax.ShapeDtypeStruct((T, D), jnp.float32),
        ],
    )(x, gate_pad)


def _dispatch(x, src_of_slot):
    mesh = plsc.VectorSubcoreMesh(core_axis_name="c", subcore_axis_name="s")

    @functools.partial(
        pl.kernel,
        out_type=jax.ShapeDtypeStruct((NSLOT, D), jnp.float32),
        mesh=mesh,
        scratch_types=[
            pltpu.VMEM((SPT,), jnp.int32),
            pltpu.VMEM((GCH, D), jnp.float32),
            pltpu.VMEM((GCH, D), jnp.float32),
            pltpu.SemaphoreType.DMA,
            pltpu.SemaphoreType.DMA,
            pltpu.SemaphoreType.DMA,
        ],
    )
    def k(x_hbm, src_hbm, xs_hbm, idx_v, buf0, buf1, gsem, wsem0, wsem1):
        w = lax.axis_index("s") * 2 + lax.axis_index("c")
        base = w * SPT
        pltpu.sync_copy(src_hbm.at[pl.ds(base, SPT)], idx_v)
        nch = SPT // GCH
        bufs = (buf0, buf1)
        wsems = (wsem0, wsem1)

        def gather(kk, buf):
            return pltpu.async_copy(
                x_hbm.at[idx_v.at[pl.ds(kk * GCH, GCH)]], buf, gsem)

        g = gather(0, bufs[0])
        for kk in range(nch):
            g.wait()
            if kk + 1 < nch:
                if kk >= 1:
                    pltpu.make_async_copy(
                        bufs[(kk + 1) % 2],
                        xs_hbm.at[pl.ds(base + (kk - 1) * GCH, GCH)],
                        wsems[(kk - 1) % 2]).wait()
                g = gather(kk + 1, bufs[(kk + 1) % 2])
            pltpu.async_copy(
                bufs[kk % 2],
                xs_hbm.at[pl.ds(base + kk * GCH, GCH)],
                wsems[kk % 2])
        pltpu.make_async_copy(
            bufs[(nch - 1) % 2],
            xs_hbm.at[pl.ds(base + (nch - 1) * GCH, GCH)],
            wsems[(nch - 1) % 2]).wait()
        pltpu.make_async_copy(
            bufs[(nch - 2) % 2],
            xs_hbm.at[pl.ds(base + (nch - 2) * GCH, GCH)],
            wsems[(nch - 2) % 2]).wait()

    return k(x, src_of_slot)


def _mlp_body(sc_ref, xs_ref, w1_ref, w3_ref, w2_ref, ws_ref, out_ref):
    e = pl.program_id(0)
    j = pl.program_id(1)

    @pl.when(j < sc_ref[e])
    def _():
        xb = xs_ref[...]
        a = lax.dot_general(xb, w1_ref[0], (((1,), (1,)), ((), ())),
                            preferred_element_type=jnp.float32)
        b = lax.dot_general(xb, w3_ref[0], (((1,), (1,)), ((), ())),
                            preferred_element_type=jnp.float32)
        h = a * jax.nn.sigmoid(a) * b
        o = lax.dot_general(h, w2_ref[0], (((1,), (1,)), ((), ())),
                            preferred_element_type=jnp.float32)
        out_ref[...] = o * ws_ref[0, 0, :][:, None]


def _mlp(scalars, xs, w1, w3, w2, w_slot3):
    def _rb(e, j, s):
        return s[E + e] + jnp.minimum(j, jnp.maximum(s[e] - 1, 0))

    grid_spec = pltpu.PrefetchScalarGridSpec(
        num_scalar_prefetch=1,
        grid=(E, JMAX),
        in_specs=[
            pl.BlockSpec((BT, D), lambda e, j, s: (_rb(e, j, s), 0)),
            pl.BlockSpec((1, F, D), lambda e, j, s: (e, 0, 0)),
            pl.BlockSpec((1, F, D), lambda e, j, s: (e, 0, 0)),
            pl.BlockSpec((1, D, F), lambda e, j, s: (e, 0, 0)),
            pl.BlockSpec((1, 1, BT), lambda e, j, s: (_rb(e, j, s), 0, 0)),
        ],
        out_specs=pl.BlockSpec((BT, D), lambda e, j, s: (_rb(e, j, s), 0)),
    )
    return pl.pallas_call(
        _mlp_body,
        grid_spec=grid_spec,
        out_shape=jax.ShapeDtypeStruct((NSLOT, D), jnp.float32),
        compiler_params=pltpu.CompilerParams(
            dimension_semantics=("arbitrary", "arbitrary")),
    )(scalars, xs, w1, w3, w2, w_slot3)


def _combine(buf, pos_a, pos_b):
    mesh = plsc.VectorSubcoreMesh(core_axis_name="c", subcore_axis_name="s")

    @functools.partial(
        pl.kernel,
        out_type=jax.ShapeDtypeStruct((T, D), jnp.float32),
        mesh=mesh,
        scratch_types=[
            pltpu.VMEM((TPT,), jnp.int32),
            pltpu.VMEM((TPT,), jnp.int32),
            pltpu.VMEM((CCH, D), jnp.float32),
            pltpu.VMEM((CCH, D), jnp.float32),
            pltpu.VMEM((CCH, D), jnp.float32),
            pltpu.SemaphoreType.DMA,
            pltpu.SemaphoreType.DMA,
        ],
    )
    def k(buf_hbm, pa_hbm, pb_hbm, out_hbm, ia_v, ib_v, a_v, b_v, o_v,
          sem, wsem):
        w = lax.axis_index("s") * 2 + lax.axis_index("c")
        base = w * TPT
        pltpu.sync_copy(pa_hbm.at[pl.ds(base, TPT)], ia_v)
        pltpu.sync_copy(pb_hbm.at[pl.ds(base, TPT)], ib_v)
        for kk in range(TPT // CCH):
            ca = pltpu.async_copy(
                buf_hbm.at[ia_v.at[pl.ds(kk * CCH, CCH)]], a_v, sem)
            cb = pltpu.async_copy(
                buf_hbm.at[ib_v.at[pl.ds(kk * CCH, CCH)]], b_v, sem)
            ca.wait()
            cb.wait()
            if kk > 0:
                pltpu.make_async_copy(
                    o_v, out_hbm.at[pl.ds(base + (kk - 1) * CCH, CCH)],
                    wsem).wait()

            @pl.loop(0, CCH)
            def _(r):
                for sg in range(D // 16):
                    sl = pl.ds(sg * 16, 16)
                    o_v[r, sl] = a_v[r, sl] + b_v[r, sl]

            pltpu.async_copy(
                o_v, out_hbm.at[pl.ds(base + kk * CCH, CCH)], wsem)
        pltpu.make_async_copy(
            o_v, out_hbm.at[pl.ds(base + (TPT // CCH - 1) * CCH, CCH)],
            wsem).wait()

    return k(buf, pos_a, pos_b)


def kernel(hidden_states, gate_w, w1, w2, w3):
    b, s, d = hidden_states.shape
    x = hidden_states.reshape(T, D)
    gate_pad = jnp.zeros((EPAD, D), jnp.float32).at[:E].set(gate_w)

    logits_pad, i1, i2, wa, wb, x_copy = _router(x, gate_pad)
    router_logits = logits_pad[:, :E]

    flat_e = jnp.stack([i1[:, 0], i2[:, 0]], axis=1).reshape(-1)
    onehot = (flat_e[:, None] == jnp.arange(E)[None, :]).astype(jnp.int32)
    csum = jnp.cumsum(onehot, axis=0)
    rank = jnp.take_along_axis(csum, flat_e[:, None], axis=1)[:, 0] - 1
    counts = csum[-1]
    nb = ((counts + BT - 1) // BT).astype(jnp.int32)
    cnb = jnp.concatenate([jnp.zeros((1,), jnp.int32),
                           jnp.cumsum(nb)[:-1].astype(jnp.int32)])
    scalars = jnp.concatenate([nb, cnb])
    pstart = cnb * BT
    pslot = (pstart[flat_e] + rank).astype(jnp.int32)
    src_tok = (jnp.arange(2 * T, dtype=jnp.int32) // 2)
    wflat = jnp.stack([wa[:, 0], wb[:, 0]], axis=1).reshape(-1)
    pad_src = (jnp.arange(NSLOT, dtype=jnp.int32) * 17) % T
    src_of_slot = pad_src.at[pslot].set(src_tok)
    w_slot = jnp.zeros((NSLOT,), jnp.float32).at[pslot].set(wflat)
    w_slot3 = w_slot.reshape(NSLOT // BT, 1, BT)
    pos_a = pslot[0::2]
    pos_b = pslot[1::2]

    xs = _dispatch(x_copy, src_of_slot)
    out_buf = _mlp(scalars, xs, w1, w3, w2, w_slot3)
    final = _combine(out_buf, pos_a, pos_b)
    return final.reshape(b, s, d), router_logits


# speedup vs baseline: 1.5925x; 1.1473x over previous
import functools

import jax
import jax.numpy as jnp
from jax import lax
from jax.experimental import pallas as pl
from jax.experimental.pallas import tpu as pltpu
from jax.experimental.pallas import tpu_sc as plsc

T = 2048
D = 1024
E = 8
F = 2048
EPAD = 128
BT = 256
JMAX = T // BT
NSLOT = 2 * T + E * BT
NBLK = NSLOT // BT
NW = 32
SPT = NSLOT // NW
GCH = 32
TPT = T // NW
CCH = 32
NEG = -1e30


def _router_body(x_ref, g_ref, logits_ref, i1_ref, i2_ref, wa_ref, wb_ref,
                 xc_ref):
    x = x_ref[...]
    xc_ref[...] = x
    g = g_ref[...]
    logits = lax.dot_general(x, g, (((1,), (1,)), ((), ())),
                             preferred_element_type=jnp.float32)
    col = lax.broadcasted_iota(jnp.int32, (T, EPAD), 1)
    lg = jnp.where(col < E, logits, NEG)
    m1 = jnp.max(lg, axis=1, keepdims=True)
    i1 = jnp.min(jnp.where(lg == m1, col, EPAD), axis=1, keepdims=True)
    lg2 = jnp.where(col == i1, NEG, lg)
    m2 = jnp.max(lg2, axis=1, keepdims=True)
    i2 = jnp.min(jnp.where(lg2 == m2, col, EPAD), axis=1, keepdims=True)
    logits_ref[...] = logits
    i1_ref[...] = i1
    i2_ref[...] = i2
    wa_ref[...] = jax.nn.sigmoid(m1 - m2)
    wb_ref[...] = jax.nn.sigmoid(m2 - m1)


def _router(x, gate_pad):
    return pl.pallas_call(
        _router_body,
        out_shape=[
            jax.ShapeDtypeStruct((T, EPAD), jnp.float32),
            jax.ShapeDtypeStruct((T, 1), jnp.int32),
            jax.ShapeDtypeStruct((T, 1), jnp.int32),
            jax.ShapeDtypeStruct((T, 1), jnp.float32),
            jax.ShapeDtypeStruct((T, 1), jnp.float32),
            jax.ShapeDtypeStruct((T, D), jnp.float32),
        ],
    )(x, gate_pad)


def _dispatch(x, src_of_slot):
    mesh = plsc.VectorSubcoreMesh(core_axis_name="c", subcore_axis_name="s")

    @functools.partial(
        pl.kernel,
        out_type=jax.ShapeDtypeStruct((NSLOT, D), jnp.float32),
        mesh=mesh,
        scratch_types=[
            pltpu.VMEM((SPT,), jnp.int32),
            pltpu.VMEM((GCH, D), jnp.float32),
            pltpu.VMEM((GCH, D), jnp.float32),
            pltpu.SemaphoreType.DMA,
            pltpu.SemaphoreType.DMA,
            pltpu.SemaphoreType.DMA,
        ],
    )
    def k(x_hbm, src_hbm, xs_hbm, idx_v, buf0, buf1, gsem, wsem0, wsem1):
        w = lax.axis_index("s") * 2 + lax.axis_index("c")
        base = w * SPT
        pltpu.sync_copy(src_hbm.at[pl.ds(base, SPT)], idx_v)
        nch = SPT // GCH
        bufs = (buf0, buf1)
        wsems = (wsem0, wsem1)

        def gather(kk, buf):
            return pltpu.async_copy(
                x_hbm.at[idx_v.at[pl.ds(kk * GCH, GCH)]], buf, gsem)

        g = gather(0, bufs[0])
        for kk in range(nch):
            g.wait()
            if kk + 1 < nch:
                if kk >= 1:
                    pltpu.make_async_copy(
                        bufs[(kk + 1) % 2],
                        xs_hbm.at[pl.ds(base + (kk - 1) * GCH, GCH)],
                        wsems[(kk - 1) % 2]).wait()
                g = gather(kk + 1, bufs[(kk + 1) % 2])
            pltpu.async_copy(
                bufs[kk % 2],
                xs_hbm.at[pl.ds(base + kk * GCH, GCH)],
                wsems[kk % 2])
        pltpu.make_async_copy(
            bufs[(nch - 1) % 2],
            xs_hbm.at[pl.ds(base + (nch - 1) * GCH, GCH)],
            wsems[(nch - 1) % 2]).wait()
        pltpu.make_async_copy(
            bufs[(nch - 2) % 2],
            xs_hbm.at[pl.ds(base + (nch - 2) * GCH, GCH)],
            wsems[(nch - 2) % 2]).wait()

    return k(x, src_of_slot)


def _mlp_body(sc_ref, xs_ref, w1_ref, w3_ref, w2_ref, ws_ref, out_ref):
    m = pl.program_id(0)

    @pl.when(m < sc_ref[NBLK])
    def _():
        xb = xs_ref[...]
        a = lax.dot_general(xb, w1_ref[0], (((1,), (1,)), ((), ())),
                            preferred_element_type=jnp.float32)
        b = lax.dot_general(xb, w3_ref[0], (((1,), (1,)), ((), ())),
                            preferred_element_type=jnp.float32)
        h = a * jax.nn.sigmoid(a) * b
        o = lax.dot_general(h, w2_ref[0], (((1,), (1,)), ((), ())),
                            preferred_element_type=jnp.float32)
        out_ref[...] = o * ws_ref[0, 0, :][:, None]


def _mlp(scalars, xs, w1, w3, w2, w_slot3):
    def _mb(m, s):
        return jnp.minimum(m, jnp.maximum(s[NBLK] - 1, 0))

    def _eb(m, s):
        return s[_mb(m, s)]

    grid_spec = pltpu.PrefetchScalarGridSpec(
        num_scalar_prefetch=1,
        grid=(NBLK,),
        in_specs=[
            pl.BlockSpec((BT, D), lambda m, s: (_mb(m, s), 0)),
            pl.BlockSpec((1, F, D), lambda m, s: (_eb(m, s), 0, 0)),
            pl.BlockSpec((1, F, D), lambda m, s: (_eb(m, s), 0, 0)),
            pl.BlockSpec((1, D, F), lambda m, s: (_eb(m, s), 0, 0)),
            pl.BlockSpec((1, 1, BT), lambda m, s: (_mb(m, s), 0, 0)),
        ],
        out_specs=pl.BlockSpec((BT, D), lambda m, s: (_mb(m, s), 0)),
    )
    return pl.pallas_call(
        _mlp_body,
        grid_spec=grid_spec,
        out_shape=jax.ShapeDtypeStruct((NSLOT, D), jnp.float32),
        compiler_params=pltpu.CompilerParams(
            dimension_semantics=("arbitrary",)),
    )(scalars, xs, w1, w3, w2, w_slot3)


def _combine(buf, pos_a, pos_b):
    mesh = plsc.VectorSubcoreMesh(core_axis_name="c", subcore_axis_name="s")

    @functools.partial(
        pl.kernel,
        out_type=jax.ShapeDtypeStruct((T, D), jnp.float32),
        mesh=mesh,
        scratch_types=[
            pltpu.VMEM((TPT,), jnp.int32),
            pltpu.VMEM((TPT,), jnp.int32),
            pltpu.VMEM((CCH, D), jnp.float32),
            pltpu.VMEM((CCH, D), jnp.float32),
            pltpu.VMEM((CCH, D), jnp.float32),
            pltpu.SemaphoreType.DMA,
            pltpu.SemaphoreType.DMA,
        ],
    )
    def k(buf_hbm, pa_hbm, pb_hbm, out_hbm, ia_v, ib_v, a_v, b_v, o_v,
          sem, wsem):
        w = lax.axis_index("s") * 2 + lax.axis_index("c")
        base = w * TPT
        pltpu.sync_copy(pa_hbm.at[pl.ds(base, TPT)], ia_v)
        pltpu.sync_copy(pb_hbm.at[pl.ds(base, TPT)], ib_v)
        for kk in range(TPT // CCH):
            ca = pltpu.async_copy(
                buf_hbm.at[ia_v.at[pl.ds(kk * CCH, CCH)]], a_v, sem)
            cb = pltpu.async_copy(
                buf_hbm.at[ib_v.at[pl.ds(kk * CCH, CCH)]], b_v, sem)
            ca.wait()
            cb.wait()
            if kk > 0:
                pltpu.make_async_copy(
                    o_v, out_hbm.at[pl.ds(base + (kk - 1) * CCH, CCH)],
                    wsem).wait()

            @pl.loop(0, CCH)
            def _(r):
                for sg in range(D // 16):
                    sl = pl.ds(sg * 16, 16)
                    o_v[r, sl] = a_v[r, sl] + b_v[r, sl]

            pltpu.async_copy(
                o_v, out_hbm.at[pl.ds(base + kk * CCH, CCH)], wsem)
        pltpu.make_async_copy(
            o_v, out_hbm.at[pl.ds(base + (TPT // CCH - 1) * CCH, CCH)],
            wsem).wait()

    return k(buf, pos_a, pos_b)


def kernel(hidden_states, gate_w, w1, w2, w3):
    b, s, d = hidden_states.shape
    x = hidden_states.reshape(T, D)
    gate_pad = jnp.zeros((EPAD, D), jnp.float32).at[:E].set(gate_w)

    logits_pad, i1, i2, wa, wb, x_copy = _router(x, gate_pad)
    router_logits = logits_pad[:, :E]

    flat_e = jnp.stack([i1[:, 0], i2[:, 0]], axis=1).reshape(-1)
    onehot = (flat_e[:, None] == jnp.arange(E)[None, :]).astype(jnp.int32)
    csum = jnp.cumsum(onehot, axis=0)
    rank = jnp.sum(csum * onehot, axis=1) - 1
    counts = csum[-1]
    nb = ((counts + BT - 1) // BT).astype(jnp.int32)
    ends = jnp.cumsum(nb)
    nblk_total = ends[-1].astype(jnp.int32)
    pstart = (ends - nb) * BT
    blk = jnp.arange(NBLK, dtype=jnp.int32)
    eb = jnp.minimum(
        jnp.sum((blk[:, None] >= ends[None, :]).astype(jnp.int32), axis=1),
        E - 1).astype(jnp.int32)
    scalars = jnp.concatenate([eb, nblk_total[None]])
    pslot = (jnp.sum(pstart[None, :] * onehot, axis=1) + rank
             ).astype(jnp.int32)
    src_tok = (jnp.arange(2 * T, dtype=jnp.int32) // 2)
    wflat = jnp.stack([wa[:, 0], wb[:, 0]], axis=1).reshape(-1)
    pad_src = ((jnp.arange(NSLOT, dtype=jnp.int32) * 17) % T
               ).astype(jnp.float32)
    init = jnp.stack([pad_src, jnp.zeros((NSLOT,), jnp.float32)], axis=1)
    merged = init.at[pslot].set(
        jnp.stack([src_tok.astype(jnp.float32), wflat], axis=1))
    src_of_slot = merged[:, 0].astype(jnp.int32)
    w_slot3 = merged[:, 1].reshape(NBLK, 1, BT)
    pos_a = pslot[0::2]
    pos_b = pslot[1::2]

    xs = _dispatch(x_copy, src_of_slot)
    out_buf = _mlp(scalars, xs, w1, w3, w2, w_slot3)
    final = _combine(out_buf, pos_a, pos_b)
    return final.reshape(b, s, d), router_logits


# speedup vs baseline: 1.6336x; 1.0258x over previous
import functools

import jax
import jax.numpy as jnp
from jax import lax
from jax.experimental import pallas as pl
from jax.experimental.pallas import tpu as pltpu
from jax.experimental.pallas import tpu_sc as plsc

T = 2048
D = 1024
E = 8
F = 2048
EPAD = 128
BT = 256
JMAX = T // BT
NSLOT = 2 * T + E * BT
NBLK = NSLOT // BT
NW = 32
SPT = NSLOT // NW
GCH = 32
TPT = T // NW
CCH = 32
NEG = -1e30


def _router_body(x_ref, g_ref, logits_ref, i1_ref, i2_ref, wa_ref, wb_ref):
    x = x_ref[...]
    g = jnp.pad(g_ref[...], ((0, EPAD - E), (0, 0)))
    logits = lax.dot_general(x, g, (((1,), (1,)), ((), ())),
                             preferred_element_type=jnp.float32)
    col = lax.broadcasted_iota(jnp.int32, (T, EPAD), 1)
    lg = jnp.where(col < E, logits, NEG)
    m1 = jnp.max(lg, axis=1, keepdims=True)
    i1 = jnp.min(jnp.where(lg == m1, col, EPAD), axis=1, keepdims=True)
    lg2 = jnp.where(col == i1, NEG, lg)
    m2 = jnp.max(lg2, axis=1, keepdims=True)
    i2 = jnp.min(jnp.where(lg2 == m2, col, EPAD), axis=1, keepdims=True)
    logits_ref[...] = logits
    i1_ref[...] = i1
    i2_ref[...] = i2
    wa_ref[...] = jax.nn.sigmoid(m1 - m2)
    wb_ref[...] = jax.nn.sigmoid(m2 - m1)


def _router(x, gate_w):
    return pl.pallas_call(
        _router_body,
        out_shape=[
            jax.ShapeDtypeStruct((T, EPAD), jnp.float32),
            jax.ShapeDtypeStruct((T, 1), jnp.int32),
            jax.ShapeDtypeStruct((T, 1), jnp.int32),
            jax.ShapeDtypeStruct((T, 1), jnp.float32),
            jax.ShapeDtypeStruct((T, 1), jnp.float32),
        ],
    )(x, gate_w)


def _dispatch(x, src_of_slot):
    mesh = plsc.VectorSubcoreMesh(core_axis_name="c", subcore_axis_name="s")

    @functools.partial(
        pl.kernel,
        out_type=jax.ShapeDtypeStruct((NSLOT, D), jnp.float32),
        mesh=mesh,
        scratch_types=[
            pltpu.VMEM((SPT,), jnp.int32),
            pltpu.VMEM((GCH, D), jnp.float32),
            pltpu.VMEM((GCH, D), jnp.float32),
            pltpu.SemaphoreType.DMA,
            pltpu.SemaphoreType.DMA,
            pltpu.SemaphoreType.DMA,
        ],
    )
    def k(x_hbm, src_hbm, xs_hbm, idx_v, buf0, buf1, gsem, wsem0, wsem1):
        w = lax.axis_index("s") * 2 + lax.axis_index("c")
        base = w * SPT
        pltpu.sync_copy(src_hbm.at[pl.ds(base, SPT)], idx_v)
        nch = SPT // GCH
        bufs = (buf0, buf1)
        wsems = (wsem0, wsem1)

        def gather(kk, buf):
            return pltpu.async_copy(
                x_hbm.at[idx_v.at[pl.ds(kk * GCH, GCH)]], buf, gsem)

        g = gather(0, bufs[0])
        for kk in range(nch):
            g.wait()
            if kk + 1 < nch:
                if kk >= 1:
                    pltpu.make_async_copy(
                        bufs[(kk + 1) % 2],
                        xs_hbm.at[pl.ds(base + (kk - 1) * GCH, GCH)],
                        wsems[(kk - 1) % 2]).wait()
                g = gather(kk + 1, bufs[(kk + 1) % 2])
            pltpu.async_copy(
                bufs[kk % 2],
                xs_hbm.at[pl.ds(base + kk * GCH, GCH)],
                wsems[kk % 2])
        pltpu.make_async_copy(
            bufs[(nch - 1) % 2],
            xs_hbm.at[pl.ds(base + (nch - 1) * GCH, GCH)],
            wsems[(nch - 1) % 2]).wait()
        pltpu.make_async_copy(
            bufs[(nch - 2) % 2],
            xs_hbm.at[pl.ds(base + (nch - 2) * GCH, GCH)],
            wsems[(nch - 2) % 2]).wait()

    return k(x, src_of_slot)


def _mlp_body(sc_ref, xs_ref, w1_ref, w3_ref, w2_ref, ws_ref, out_ref):
    m = pl.program_id(0)

    @pl.when(m < sc_ref[NBLK])
    def _():
        xb = xs_ref[...]
        a = lax.dot_general(xb, w1_ref[0], (((1,), (1,)), ((), ())),
                            preferred_element_type=jnp.float32)
        b = lax.dot_general(xb, w3_ref[0], (((1,), (1,)), ((), ())),
                            preferred_element_type=jnp.float32)
        h = a * jax.nn.sigmoid(a) * b
        o = lax.dot_general(h, w2_ref[0], (((1,), (1,)), ((), ())),
                            preferred_element_type=jnp.float32)
        out_ref[...] = o * ws_ref[0, 0, :][:, None]


def _mlp(scalars, xs, w1, w3, w2, w_slot3):
    def _mb(m, s):
        return jnp.minimum(m, jnp.maximum(s[NBLK] - 1, 0))

    def _eb(m, s):
        return s[_mb(m, s)]

    grid_spec = pltpu.PrefetchScalarGridSpec(
        num_scalar_prefetch=1,
        grid=(NBLK,),
        in_specs=[
            pl.BlockSpec((BT, D), lambda m, s: (_mb(m, s), 0)),
            pl.BlockSpec((1, F, D), lambda m, s: (_eb(m, s), 0, 0)),
            pl.BlockSpec((1, F, D), lambda m, s: (_eb(m, s), 0, 0)),
            pl.BlockSpec((1, D, F), lambda m, s: (_eb(m, s), 0, 0)),
            pl.BlockSpec((1, 1, BT), lambda m, s: (_mb(m, s), 0, 0)),
        ],
        out_specs=pl.BlockSpec((BT, D), lambda m, s: (_mb(m, s), 0)),
    )
    return pl.pallas_call(
        _mlp_body,
        grid_spec=grid_spec,
        out_shape=jax.ShapeDtypeStruct((NSLOT, D), jnp.float32),
        compiler_params=pltpu.CompilerParams(
            dimension_semantics=("arbitrary",)),
    )(scalars, xs, w1, w3, w2, w_slot3)


def _combine(buf, pos_a, pos_b):
    mesh = plsc.VectorSubcoreMesh(core_axis_name="c", subcore_axis_name="s")

    @functools.partial(
        pl.kernel,
        out_type=jax.ShapeDtypeStruct((T, D), jnp.float32),
        mesh=mesh,
        scratch_types=[
            pltpu.VMEM((TPT,), jnp.int32),
            pltpu.VMEM((TPT,), jnp.int32),
            pltpu.VMEM((CCH, D), jnp.float32),
            pltpu.VMEM((CCH, D), jnp.float32),
            pltpu.VMEM((CCH, D), jnp.float32),
            pltpu.SemaphoreType.DMA,
            pltpu.SemaphoreType.DMA,
        ],
    )
    def k(buf_hbm, pa_hbm, pb_hbm, out_hbm, ia_v, ib_v, a_v, b_v, o_v,
          sem, wsem):
        w = lax.axis_index("s") * 2 + lax.axis_index("c")
        base = w * TPT
        pltpu.sync_copy(pa_hbm.at[pl.ds(base, TPT)], ia_v)
        pltpu.sync_copy(pb_hbm.at[pl.ds(base, TPT)], ib_v)
        for kk in range(TPT // CCH):
            ca = pltpu.async_copy(
                buf_hbm.at[ia_v.at[pl.ds(kk * CCH, CCH)]], a_v, sem)
            cb = pltpu.async_copy(
                buf_hbm.at[ib_v.at[pl.ds(kk * CCH, CCH)]], b_v, sem)
            ca.wait()
            cb.wait()
            if kk > 0:
                pltpu.make_async_copy(
                    o_v, out_hbm.at[pl.ds(base + (kk - 1) * CCH, CCH)],
                    wsem).wait()

            @pl.loop(0, CCH)
            def _(r):
                for sg in range(D // 16):
                    sl = pl.ds(sg * 16, 16)
                    o_v[r, sl] = a_v[r, sl] + b_v[r, sl]

            pltpu.async_copy(
                o_v, out_hbm.at[pl.ds(base + kk * CCH, CCH)], wsem)
        pltpu.make_async_copy(
            o_v, out_hbm.at[pl.ds(base + (TPT // CCH - 1) * CCH, CCH)],
            wsem).wait()

    return k(buf, pos_a, pos_b)


def kernel(hidden_states, gate_w, w1, w2, w3):
    b, s, d = hidden_states.shape
    x = hidden_states.reshape(T, D)

    logits_pad, i1, i2, wa, wb = _router(x, gate_w)
    router_logits = logits_pad[:, :E]

    eye = jnp.arange(E, dtype=jnp.int32)[None, :]
    oh1 = (i1 == eye).astype(jnp.int32)
    oh2 = (i2 == eye).astype(jnp.int32)
    oh12 = oh1 + oh2
    csum = jnp.cumsum(oh12, axis=0)
    cexcl = csum - oh12
    counts = csum[-1]
    nb = ((counts + BT - 1) // BT).astype(jnp.int32)
    ends = jnp.cumsum(nb)
    nblk_total = ends[-1].astype(jnp.int32)
    pstart = (ends - nb) * BT
    blk = jnp.arange(NBLK, dtype=jnp.int32)
    eb = jnp.minimum(
        jnp.sum((blk[:, None] >= ends[None, :]).astype(jnp.int32), axis=1),
        E - 1).astype(jnp.int32)
    scalars = jnp.concatenate([eb, nblk_total[None]])
    pos_a = (jnp.sum((pstart[None, :] + cexcl) * oh1, axis=1)
             ).astype(jnp.int32)
    pos_b = (jnp.sum((pstart[None, :] + cexcl) * oh2, axis=1)
             ).astype(jnp.int32)
    tokf = jnp.arange(T, dtype=jnp.float32)
    pad_src = ((jnp.arange(NSLOT, dtype=jnp.int32) * 17) % T
               ).astype(jnp.float32)
    init = jnp.stack([pad_src, jnp.zeros((NSLOT,), jnp.float32)], axis=1)
    merged = init.at[jnp.concatenate([pos_a, pos_b])].set(
        jnp.stack([jnp.concatenate([tokf, tokf]),
                   jnp.concatenate([wa[:, 0], wb[:, 0]])], axis=1))
    src_of_slot = merged[:, 0].astype(jnp.int32)
    w_slot3 = merged[:, 1].reshape(NBLK, 1, BT)

    xs = _dispatch(x, src_of_slot)
    out_buf = _mlp(scalars, xs, w1, w3, w2, w_slot3)
    final = _combine(out_buf, pos_a, pos_b)
    return final.reshape(b, s, d), router_logits


# speedup vs baseline: 1.6810x; 1.0291x over previous
import functools

import jax
import jax.numpy as jnp
from jax import lax
from jax.experimental import pallas as pl
from jax.experimental.pallas import tpu as pltpu
from jax.experimental.pallas import tpu_sc as plsc

T = 2048
D = 1024
E = 8
F = 2048
EPAD = 128
BT = 256
JMAX = T // BT
NSLOT = 2 * T + E * BT
NBLK = NSLOT // BT
NW = 32
SPT = NSLOT // NW
GCH = 48
TPT = T // NW
CCH = 32
NEG = -1e30


def _router_body(x_ref, g_ref, logits_ref, i1_ref, i2_ref, wa_ref, wb_ref):
    x = x_ref[...]
    g = jnp.pad(g_ref[...], ((0, EPAD - E), (0, 0)))
    logits = lax.dot_general(x, g, (((1,), (1,)), ((), ())),
                             preferred_element_type=jnp.float32)
    col = lax.broadcasted_iota(jnp.int32, (T, EPAD), 1)
    lg = jnp.where(col < E, logits, NEG)
    m1 = jnp.max(lg, axis=1, keepdims=True)
    i1 = jnp.min(jnp.where(lg == m1, col, EPAD), axis=1, keepdims=True)
    lg2 = jnp.where(col == i1, NEG, lg)
    m2 = jnp.max(lg2, axis=1, keepdims=True)
    i2 = jnp.min(jnp.where(lg2 == m2, col, EPAD), axis=1, keepdims=True)
    logits_ref[...] = logits
    i1_ref[...] = i1
    i2_ref[...] = i2
    wa_ref[...] = jax.nn.sigmoid(m1 - m2)
    wb_ref[...] = jax.nn.sigmoid(m2 - m1)


def _router(x, gate_w):
    return pl.pallas_call(
        _router_body,
        out_shape=[
            jax.ShapeDtypeStruct((T, EPAD), jnp.float32),
            jax.ShapeDtypeStruct((T, 1), jnp.int32),
            jax.ShapeDtypeStruct((T, 1), jnp.int32),
            jax.ShapeDtypeStruct((T, 1), jnp.float32),
            jax.ShapeDtypeStruct((T, 1), jnp.float32),
        ],
    )(x, gate_w)


def _dispatch(x, src_of_slot):
    mesh = plsc.VectorSubcoreMesh(core_axis_name="c", subcore_axis_name="s")

    @functools.partial(
        pl.kernel,
        out_type=jax.ShapeDtypeStruct((NSLOT, D), jnp.float32),
        mesh=mesh,
        scratch_types=[
            pltpu.VMEM((SPT,), jnp.int32),
            pltpu.VMEM((GCH, D), jnp.float32),
            pltpu.VMEM((GCH, D), jnp.float32),
            pltpu.SemaphoreType.DMA,
            pltpu.SemaphoreType.DMA,
            pltpu.SemaphoreType.DMA,
        ],
    )
    def k(x_hbm, src_hbm, xs_hbm, idx_v, buf0, buf1, gsem, wsem0, wsem1):
        w = lax.axis_index("s") * 2 + lax.axis_index("c")
        base = w * SPT
        pltpu.sync_copy(src_hbm.at[pl.ds(base, SPT)], idx_v)
        nch = SPT // GCH
        bufs = (buf0, buf1)
        wsems = (wsem0, wsem1)

        def gather(kk, buf):
            return pltpu.async_copy(
                x_hbm.at[idx_v.at[pl.ds(kk * GCH, GCH)]], buf, gsem)

        g = gather(0, bufs[0])
        for kk in range(nch):
            g.wait()
            if kk + 1 < nch:
                if kk >= 1:
                    pltpu.make_async_copy(
                        bufs[(kk + 1) % 2],
                        xs_hbm.at[pl.ds(base + (kk - 1) * GCH, GCH)],
                        wsems[(kk - 1) % 2]).wait()
                g = gather(kk + 1, bufs[(kk + 1) % 2])
            pltpu.async_copy(
                bufs[kk % 2],
                xs_hbm.at[pl.ds(base + kk * GCH, GCH)],
                wsems[kk % 2])
        pltpu.make_async_copy(
            bufs[(nch - 1) % 2],
            xs_hbm.at[pl.ds(base + (nch - 1) * GCH, GCH)],
            wsems[(nch - 1) % 2]).wait()
        pltpu.make_async_copy(
            bufs[(nch - 2) % 2],
            xs_hbm.at[pl.ds(base + (nch - 2) * GCH, GCH)],
            wsems[(nch - 2) % 2]).wait()

    return k(x, src_of_slot)


def _mlp_body(sc_ref, xs_ref, w1_ref, w3_ref, w2_ref, ws_ref, out_ref):
    m = pl.program_id(0)

    @pl.when(m < sc_ref[NBLK])
    def _():
        xb = xs_ref[...]
        a = lax.dot_general(xb, w1_ref[0], (((1,), (1,)), ((), ())),
                            preferred_element_type=jnp.float32)
        b = lax.dot_general(xb, w3_ref[0], (((1,), (1,)), ((), ())),
                            preferred_element_type=jnp.float32)
        h = a * jax.nn.sigmoid(a) * b
        o = lax.dot_general(h, w2_ref[0], (((1,), (1,)), ((), ())),
                            preferred_element_type=jnp.float32)
        out_ref[...] = o * ws_ref[0, 0, :][:, None]


def _mlp(scalars, xs, w1, w3, w2, w_slot3):
    def _mb(m, s):
        return jnp.minimum(m, jnp.maximum(s[NBLK] - 1, 0))

    def _eb(m, s):
        return s[_mb(m, s)]

    grid_spec = pltpu.PrefetchScalarGridSpec(
        num_scalar_prefetch=1,
        grid=(NBLK,),
        in_specs=[
            pl.BlockSpec((BT, D), lambda m, s: (_mb(m, s), 0)),
            pl.BlockSpec((1, F, D), lambda m, s: (_eb(m, s), 0, 0)),
            pl.BlockSpec((1, F, D), lambda m, s: (_eb(m, s), 0, 0)),
            pl.BlockSpec((1, D, F), lambda m, s: (_eb(m, s), 0, 0)),
            pl.BlockSpec((1, 1, BT), lambda m, s: (_mb(m, s), 0, 0)),
        ],
        out_specs=pl.BlockSpec((BT, D), lambda m, s: (_mb(m, s), 0)),
    )
    return pl.pallas_call(
        _mlp_body,
        grid_spec=grid_spec,
        out_shape=jax.ShapeDtypeStruct((NSLOT, D), jnp.float32),
        compiler_params=pltpu.CompilerParams(
            dimension_semantics=("arbitrary",)),
    )(scalars, xs, w1, w3, w2, w_slot3)


def _combine(buf, pos_a, pos_b):
    mesh = plsc.VectorSubcoreMesh(core_axis_name="c", subcore_axis_name="s")

    @functools.partial(
        pl.kernel,
        out_type=jax.ShapeDtypeStruct((T, D), jnp.float32),
        mesh=mesh,
        scratch_types=[
            pltpu.VMEM((TPT,), jnp.int32),
            pltpu.VMEM((TPT,), jnp.int32),
            pltpu.VMEM((CCH, D), jnp.float32),
            pltpu.VMEM((CCH, D), jnp.float32),
            pltpu.VMEM((CCH, D), jnp.float32),
            pltpu.SemaphoreType.DMA,
            pltpu.SemaphoreType.DMA,
        ],
    )
    def k(buf_hbm, pa_hbm, pb_hbm, out_hbm, ia_v, ib_v, a_v, b_v, o_v,
          sem, wsem):
        w = lax.axis_index("s") * 2 + lax.axis_index("c")
        base = w * TPT
        pltpu.sync_copy(pa_hbm.at[pl.ds(base, TPT)], ia_v)
        pltpu.sync_copy(pb_hbm.at[pl.ds(base, TPT)], ib_v)
        for kk in range(TPT // CCH):
            ca = pltpu.async_copy(
                buf_hbm.at[ia_v.at[pl.ds(kk * CCH, CCH)]], a_v, sem)
            cb = pltpu.async_copy(
                buf_hbm.at[ib_v.at[pl.ds(kk * CCH, CCH)]], b_v, sem)
            ca.wait()
            cb.wait()
            if kk > 0:
                pltpu.make_async_copy(
                    o_v, out_hbm.at[pl.ds(base + (kk - 1) * CCH, CCH)],
                    wsem).wait()

            @pl.loop(0, CCH)
            def _(r):
                for sg in range(D // 16):
                    sl = pl.ds(sg * 16, 16)
                    o_v[r, sl] = a_v[r, sl] + b_v[r, sl]

            pltpu.async_copy(
                o_v, out_hbm.at[pl.ds(base + kk * CCH, CCH)], wsem)
        pltpu.make_async_copy(
            o_v, out_hbm.at[pl.ds(base + (TPT // CCH - 1) * CCH, CCH)],
            wsem).wait()

    return k(buf, pos_a, pos_b)


def kernel(hidden_states, gate_w, w1, w2, w3):
    b, s, d = hidden_states.shape
    x = hidden_states.reshape(T, D)

    logits_pad, i1, i2, wa, wb = _router(x, gate_w)
    router_logits = logits_pad[:, :E]

    eye = jnp.arange(E, dtype=jnp.int32)[None, :]
    oh1 = (i1 == eye).astype(jnp.int32)
    oh2 = (i2 == eye).astype(jnp.int32)
    oh12 = oh1 + oh2
    csum = jnp.cumsum(oh12, axis=0)
    cexcl = csum - oh12
    counts = csum[-1]
    nb = ((counts + BT - 1) // BT).astype(jnp.int32)
    ends = jnp.cumsum(nb)
    nblk_total = ends[-1].astype(jnp.int32)
    pstart = (ends - nb) * BT
    blk = jnp.arange(NBLK, dtype=jnp.int32)
    eb = jnp.minimum(
        jnp.sum((blk[:, None] >= ends[None, :]).astype(jnp.int32), axis=1),
        E - 1).astype(jnp.int32)
    scalars = jnp.concatenate([eb, nblk_total[None]])
    pos_a = (jnp.sum((pstart[None, :] + cexcl) * oh1, axis=1)
             ).astype(jnp.int32)
    pos_b = (jnp.sum((pstart[None, :] + cexcl) * oh2, axis=1)
             ).astype(jnp.int32)
    tok = jnp.arange(T, dtype=jnp.int32)
    pall = jnp.concatenate([pos_a, pos_b])
    pad_src = (jnp.arange(NSLOT, dtype=jnp.int32) * 17) % T
    src_of_slot = pad_src.at[pall].set(jnp.concatenate([tok, tok]))
    xs = _dispatch(x, src_of_slot)
    w_slot = jnp.zeros((NSLOT,), jnp.float32).at[pall].set(
        jnp.concatenate([wa[:, 0], wb[:, 0]]))
    w_slot3 = w_slot.reshape(NBLK, 1, BT)
    out_buf = _mlp(scalars, xs, w1, w3, w2, w_slot3)
    final = _combine(out_buf, pos_a, pos_b)
    return final.reshape(b, s, d), router_logits
